# Initial kernel scaffold; baseline (speedup 1.0000x reference)
import functools

import jax
import jax.numpy as jnp
from jax import lax
from jax.experimental import pallas as pl
from jax.experimental.pallas import tpu as pltpu
from jax.experimental.pallas import tpu_sc as plsc

_N_TOKEN = 1000000
_D = 128
_CUT = (0, 20000, 100000, 500000, 1000000)
_SCALE = float(_D) ** 0.5

_BLK = 4000

_NC = 2
_NS = 16
_NW = _NC * _NS
_B_TOTAL = 4096 * 200
_BPW = _B_TOTAL // _NW
_CH = 128
_NCHUNK = _BPW // _CH
_NBUF = 4
_NOUT = _NCHUNK // _NBUF


def _proj_body(emb_ref, pjt_ref, u_ref, out_ref):
    del u_ref
    out_ref[...] = (
        jnp.dot(emb_ref[...], pjt_ref[...], preferred_element_type=jnp.float32)
        * _SCALE
    )


def _project_cluster(emb, pjt, unified, row_off):
    rows, d = emb.shape
    grid = (rows // _BLK,)
    off_blocks = row_off // _BLK
    return pl.pallas_call(
        _proj_body,
        grid=grid,
        in_specs=[
            pl.BlockSpec((_BLK, d), lambda b: (b, 0)),
            pl.BlockSpec((d, _D), lambda b: (0, 0)),
            pl.BlockSpec(memory_space=pltpu.ANY),
        ],
        out_specs=pl.BlockSpec((_BLK, _D), lambda b, _o=off_blocks: (b + _o, 0)),
        out_shape=jax.ShapeDtypeStruct((_N_TOKEN, _D), jnp.float32),
        input_output_aliases={2: 0},
    )(emb, pjt, unified)


def _proj_body_first(emb_ref, pjt_ref, out_ref):
    out_ref[...] = (
        jnp.dot(emb_ref[...], pjt_ref[...], preferred_element_type=jnp.float32)
        * _SCALE
    )


def _project_first(emb, pjt):
    rows, d = emb.shape
    return pl.pallas_call(
        _proj_body_first,
        grid=(rows // _BLK,),
        in_specs=[
            pl.BlockSpec((_BLK, d), lambda b: (b, 0)),
            pl.BlockSpec((d, _D), lambda b: (0, 0)),
        ],
        out_specs=pl.BlockSpec((_BLK, _D), lambda b: (b, 0)),
        out_shape=jax.ShapeDtypeStruct((_N_TOKEN, _D), jnp.float32),
    )(emb, pjt)


def _gather_body(tab_hbm, idx_hbm, out_hbm, idx_v, rows_v, *sems):
    gsem = sems[:_NBUF]
    wsem = sems[_NBUF:]
    wid = lax.axis_index("s") * _NC + lax.axis_index("c")
    base = wid * _BPW

    pltpu.sync_copy(idx_hbm.at[pl.ds(base, _BPW)], idx_v)

    def g_copy(c, b):
        return pltpu.make_async_copy(
            tab_hbm.at[idx_v.at[pl.ds(c * _CH, _CH)]], rows_v.at[b], gsem[b]
        )

    def w_copy(c, b):
        return pltpu.make_async_copy(
            rows_v.at[b], out_hbm.at[pl.ds(base + c * _CH, _CH)], wsem[b]
        )

    for b in range(_NBUF):
        g_copy(b, b).start()

    def outer(g, carry):
        for b in range(_NBUF):
            c = g * _NBUF + b
            g_copy(c, b).wait()
            wc = w_copy(c, b)
            wc.start()
            wc.wait()
            g_copy(c + _NBUF, b).start()
        return carry

    lax.fori_loop(0, _NOUT - 1, outer, 0)

    for b in range(_NBUF):
        c = (_NOUT - 1) * _NBUF + b
        g_copy(c, b).wait()
        wc = w_copy(c, b)
        wc.start()
        wc.wait()


def _sc_gather(unified, idx):
    mesh = plsc.VectorSubcoreMesh(core_axis_name="c", subcore_axis_name="s")
    scratch = [
        pltpu.VMEM((_BPW,), jnp.int32),
        pltpu.VMEM((_NBUF, _CH, _D), jnp.float32),
    ] + [pltpu.SemaphoreType.DMA] * (2 * _NBUF)
    run = pl.kernel(
        _gather_body,
        out_type=jax.ShapeDtypeStruct((_B_TOTAL, _D), jnp.float32),
        mesh=mesh,
        scratch_types=scratch,
    )
    return run(unified, idx)


def kernel(inp, emb0, emb1, emb2, emb3, proj0, proj1, proj2, proj3):
    idx = inp.reshape(-1).astype(jnp.int32)
    unified = _project_first(emb0, proj0.T)
    unified = _project_cluster(emb1, proj1.T, unified, _CUT[1])
    unified = _project_cluster(emb2, proj2.T, unified, _CUT[2])
    unified = _project_cluster(emb3, proj3.T, unified, _CUT[3])
    out = _sc_gather(unified, idx)
    return out.reshape(inp.shape + (_D,))


# speedup vs baseline: 46.4789x; 46.4789x over previous
import functools

import jax
import jax.numpy as jnp
from jax import lax
from jax.experimental import pallas as pl
from jax.experimental.pallas import tpu as pltpu
from jax.experimental.pallas import tpu_sc as plsc

_N_TOKEN = 1000000
_D = 128
_CUT = (0, 20000, 100000, 500000, 1000000)
_SCALE = float(_D) ** 0.5

_BLK = 4000

_NC = 2
_NS = 16
_NW = _NC * _NS
_B_TOTAL = 4096 * 200
_BPW = _B_TOTAL // _NW
_CH = 128
_NCHUNK = _BPW // _CH
_NBUF = 4
_NOUT = _NCHUNK // _NBUF


def _proj_body(emb_ref, pjt_ref, u_ref, out_ref):
    del u_ref
    out_ref[...] = (
        jnp.dot(emb_ref[...], pjt_ref[...], preferred_element_type=jnp.float32)
        * _SCALE
    )


def _project_cluster(emb, pjt, unified, row_off):
    rows, d = emb.shape
    grid = (rows // _BLK,)
    off_blocks = row_off // _BLK
    return pl.pallas_call(
        _proj_body,
        grid=grid,
        in_specs=[
            pl.BlockSpec((_BLK, d), lambda b: (b, 0)),
            pl.BlockSpec((d, _D), lambda b: (0, 0)),
            pl.BlockSpec(memory_space=pltpu.MemorySpace.HBM),
        ],
        out_specs=pl.BlockSpec((_BLK, _D), lambda b, _o=off_blocks: (b + _o, 0)),
        out_shape=jax.ShapeDtypeStruct((_N_TOKEN, _D), jnp.float32),
        input_output_aliases={2: 0},
    )(emb, pjt, unified)


def _proj_body_first(emb_ref, pjt_ref, out_ref):
    out_ref[...] = (
        jnp.dot(emb_ref[...], pjt_ref[...], preferred_element_type=jnp.float32)
        * _SCALE
    )


def _project_first(emb, pjt):
    rows, d = emb.shape
    return pl.pallas_call(
        _proj_body_first,
        grid=(rows // _BLK,),
        in_specs=[
            pl.BlockSpec((_BLK, d), lambda b: (b, 0)),
            pl.BlockSpec((d, _D), lambda b: (0, 0)),
        ],
        out_specs=pl.BlockSpec((_BLK, _D), lambda b: (b, 0)),
        out_shape=jax.ShapeDtypeStruct((_N_TOKEN, _D), jnp.float32),
    )(emb, pjt)


def _gather_body(tab_hbm, idx_hbm, out_hbm, idx_v, rows_v, *sems):
    gsem = sems[:_NBUF]
    wsem = sems[_NBUF:]
    wid = lax.axis_index("s") * _NC + lax.axis_index("c")
    base = wid * _BPW

    pltpu.sync_copy(idx_hbm.at[pl.ds(base, _BPW)], idx_v)

    def g_copy(c, b):
        return pltpu.make_async_copy(
            tab_hbm.at[idx_v.at[pl.ds(c * _CH, _CH)]], rows_v.at[b], gsem[b]
        )

    def w_copy(c, b):
        return pltpu.make_async_copy(
            rows_v.at[b], out_hbm.at[pl.ds(base + c * _CH, _CH)], wsem[b]
        )

    for b in range(_NBUF):
        g_copy(b, b).start()

    def outer(g, carry):
        for b in range(_NBUF):
            c = g * _NBUF + b
            g_copy(c, b).wait()
            wc = w_copy(c, b)
            wc.start()
            wc.wait()
            g_copy(c + _NBUF, b).start()
        return carry

    lax.fori_loop(0, _NOUT - 1, outer, 0)

    for b in range(_NBUF):
        c = (_NOUT - 1) * _NBUF + b
        g_copy(c, b).wait()
        wc = w_copy(c, b)
        wc.start()
        wc.wait()


def _sc_gather(unified, idx):
    mesh = plsc.VectorSubcoreMesh(
        core_axis_name="c", subcore_axis_name="s",
        num_cores=_NC, num_subcores=_NS,
    )
    scratch = [
        pltpu.VMEM((_BPW,), jnp.int32),
        pltpu.VMEM((_NBUF, _CH, _D), jnp.float32),
    ] + [pltpu.SemaphoreType.DMA] * (2 * _NBUF)
    run = pl.kernel(
        _gather_body,
        out_type=jax.ShapeDtypeStruct((_B_TOTAL, _D), jnp.float32),
        mesh=mesh,
        scratch_types=scratch,
    )
    return run(unified, idx)


def kernel(inp, emb0, emb1, emb2, emb3, proj0, proj1, proj2, proj3):
    idx = inp.reshape(-1).astype(jnp.int32)
    unified = _project_first(emb0, proj0.T)
    unified = _project_cluster(emb1, proj1.T, unified, _CUT[1])
    unified = _project_cluster(emb2, proj2.T, unified, _CUT[2])
    unified = _project_cluster(emb3, proj3.T, unified, _CUT[3])
    out = _sc_gather(unified, idx)
    return out.reshape(inp.shape + (_D,))
